# baseline (speedup 1.0000x reference)
import jax
import jax.numpy as jnp
from jax.experimental import pallas as pl
from jax.experimental.pallas import tpu as pltpu

M = 16384
N = 4096
C = 128
CS = 64
BQ = 256
NBLK = M // BQ


def _fp_block(pos_skip_ref, x_skip_ref, posT_ref, x_ref, w_ref, b_ref,
              out_ref):
    q = pos_skip_ref[...]
    p = posT_ref[...]
    qsq = jnp.sum(q * q, axis=1, keepdims=True)
    psq = jnp.sum(p * p, axis=0, keepdims=True)
    cross = jax.lax.dot_general(
        q, p, (((1,), (0,)), ((), ())),
        preferred_element_type=jnp.float32)
    d2 = (qsq + psq) - (cross + cross)

    inf = jnp.float32(jnp.inf)
    sel_w = jnp.zeros((BQ, N), jnp.float32)
    wsum = jnp.zeros((BQ, 1), jnp.float32)
    for _ in range(3):
        m = jnp.min(d2, axis=1, keepdims=True)
        hit = d2 == m
        w = 1.0 / jnp.maximum(m, 1e-16)
        sel_w = jnp.where(hit, w, sel_w)
        wsum = wsum + w
        d2 = jnp.where(hit, inf, d2)

    num = jax.lax.dot_general(
        sel_w, x_ref[...], (((1,), (0,)), ((), ())),
        preferred_element_type=jnp.float32,
        precision=jax.lax.Precision.HIGHEST)
    h = jnp.concatenate([num / wsum, x_skip_ref[...]], axis=1)
    out_ref[...] = jax.lax.dot_general(
        h, w_ref[...], (((1,), (1,)), ((), ())),
        preferred_element_type=jnp.float32) + b_ref[...]


def kernel(x, pos, batch, x_skip, pos_skip, batch_skip, W, b):
    posT = pos.T.astype(jnp.bfloat16).astype(jnp.float32)
    ps_r = pos_skip.astype(jnp.bfloat16).astype(jnp.float32)
    b2 = b.reshape(1, C)

    out = pl.pallas_call(
        _fp_block,
        grid=(NBLK,),
        in_specs=[
            pl.BlockSpec((BQ, 3), lambda i: (i, 0)),
            pl.BlockSpec((BQ, CS), lambda i: (i, 0)),
            pl.BlockSpec((3, N), lambda i: (0, 0)),
            pl.BlockSpec((N, C), lambda i: (0, 0)),
            pl.BlockSpec((C, C + CS), lambda i: (0, 0)),
            pl.BlockSpec((1, C), lambda i: (0, 0)),
        ],
        out_specs=pl.BlockSpec((BQ, C), lambda i: (i, 0)),
        out_shape=jax.ShapeDtypeStruct((M, C), jnp.float32),
    )(ps_r, x_skip, posT, x, W, b2)

    return (out, pos_skip, batch_skip)


# speedup vs baseline: 1.5501x; 1.5501x over previous
import jax
import jax.numpy as jnp
from jax.experimental import pallas as pl
from jax.experimental.pallas import tpu as pltpu

M = 16384
N = 4096
C = 128
CS = 64
BQ = 256
NBLK = M // BQ


def _fp_block(pos_skip_ref, x_skip_ref, posT_ref, xhi_ref, xlo_ref, w_ref,
              b_ref, out_ref):
    q = pos_skip_ref[...]
    p = posT_ref[...]
    qsq = jnp.sum(q * q, axis=1, keepdims=True)
    psq = jnp.sum(p * p, axis=0, keepdims=True)
    cross = jax.lax.dot_general(
        q, p, (((1,), (0,)), ((), ())),
        preferred_element_type=jnp.float32)
    d2 = (qsq + psq) - (cross + cross)

    inf = jnp.float32(jnp.inf)
    sel_w = jnp.zeros((BQ, N), jnp.float32)
    wsum = jnp.zeros((BQ, 1), jnp.float32)
    for _ in range(3):
        m = jnp.min(d2, axis=1, keepdims=True)
        hit = d2 == m
        w = 1.0 / jnp.maximum(m, 1e-16)
        sel_w = jnp.where(hit, w, sel_w)
        wsum = wsum + w
        d2 = jnp.where(hit, inf, d2)

    dims = (((1,), (0,)), ((), ()))
    s_hi = sel_w.astype(jnp.bfloat16)
    s_lo = (sel_w - s_hi.astype(jnp.float32)).astype(jnp.bfloat16)
    num = (jax.lax.dot_general(s_hi, xhi_ref[...], dims,
                               preferred_element_type=jnp.float32)
           + jax.lax.dot_general(s_hi, xlo_ref[...], dims,
                                 preferred_element_type=jnp.float32)
           + jax.lax.dot_general(s_lo, xhi_ref[...], dims,
                                 preferred_element_type=jnp.float32))
    h = jnp.concatenate([num / wsum, x_skip_ref[...]], axis=1)
    out_ref[...] = jax.lax.dot_general(
        h, w_ref[...], (((1,), (1,)), ((), ())),
        preferred_element_type=jnp.float32) + b_ref[...]


def kernel(x, pos, batch, x_skip, pos_skip, batch_skip, W, b):
    posT = pos.T.astype(jnp.bfloat16).astype(jnp.float32)
    ps_r = pos_skip.astype(jnp.bfloat16).astype(jnp.float32)
    x_hi = x.astype(jnp.bfloat16)
    x_lo = (x - x_hi.astype(jnp.float32)).astype(jnp.bfloat16)
    b2 = b.reshape(1, C)

    out = pl.pallas_call(
        _fp_block,
        grid=(NBLK,),
        in_specs=[
            pl.BlockSpec((BQ, 3), lambda i: (i, 0)),
            pl.BlockSpec((BQ, CS), lambda i: (i, 0)),
            pl.BlockSpec((3, N), lambda i: (0, 0)),
            pl.BlockSpec((N, C), lambda i: (0, 0)),
            pl.BlockSpec((N, C), lambda i: (0, 0)),
            pl.BlockSpec((C, C + CS), lambda i: (0, 0)),
            pl.BlockSpec((1, C), lambda i: (0, 0)),
        ],
        out_specs=pl.BlockSpec((BQ, C), lambda i: (i, 0)),
        out_shape=jax.ShapeDtypeStruct((M, C), jnp.float32),
    )(ps_r, x_skip, posT, x_hi, x_lo, W, b2)

    return (out, pos_skip, batch_skip)
